# Initial kernel scaffold; baseline (speedup 1.0000x reference)
import jax
import jax.numpy as jnp
from jax.experimental import pallas as pl


def kernel(x, u, cap, logits, logit_scale, cap_scale):
    raise NotImplementedError("write your pallas kernel here")


# speedup vs baseline: 20.8502x; 20.8502x over previous
import functools

import jax
import jax.numpy as jnp
from jax.experimental import pallas as pl

_B, _T, _D = 4, 4096, 2048
_G, _CG, _K = 16, 128, 16
_TB = 256


def _body(a_ref, b_ref, x_ref, u_ref, cap_ref, o_ref):
    b = b_ref[0, 0]
    s = cap_ref[...] * b + a_ref[...]
    cur = s
    removed = jnp.zeros((_TB, _G, 1), jnp.float32)
    thresh = jnp.full((_TB, _G, 1), -jnp.inf, jnp.float32)
    kf = jnp.float32(_K)
    for _ in range(_K):
        m = jnp.max(cur, axis=-1, keepdims=True)
        eq = cur == m
        c = jnp.sum(eq.astype(jnp.float32), axis=-1, keepdims=True)
        nr = removed + c
        setm = jnp.logical_and(removed < kf, nr >= kf)
        thresh = jnp.where(setm, m, thresh)
        cur = jnp.where(eq, -jnp.inf, cur)
        removed = nr
    gated = jnp.where(s >= thresh, s, jnp.float32(-1e9))
    g = jax.nn.sigmoid(gated)
    g = g * g
    o_ref[...] = x_ref[...] + g * u_ref[...]


@jax.jit
def kernel(x, u, cap, logits, logit_scale, cap_scale):
    n = _B * _T
    x3 = x.reshape(n, _G, _CG)
    u3 = u.reshape(n, _G, _CG)
    cap3 = cap.reshape(n, _G, _CG)
    a = (logit_scale * logits).reshape(1, _G, _CG).astype(jnp.float32)
    b = (logit_scale * cap_scale).reshape(1, 1).astype(jnp.float32)

    grid = (n // _TB,)
    blk = pl.BlockSpec((_TB, _G, _CG), lambda i: (i, 0, 0))
    out = pl.pallas_call(
        _body,
        grid=grid,
        in_specs=[
            pl.BlockSpec((1, _G, _CG), lambda i: (0, 0, 0)),
            pl.BlockSpec((1, 1), lambda i: (0, 0)),
            blk,
            blk,
            blk,
        ],
        out_specs=blk,
        out_shape=jax.ShapeDtypeStruct((n, _G, _CG), jnp.float32),
    )(a, b, x3, u3, cap3)
    return out.reshape(_B, _T, _D)
